# baseline (speedup 1.0000x reference)
import jax
import jax.numpy as jnp
from jax.experimental import pallas as pl
from jax.experimental.pallas import tpu as pltpu

_N, _D, _H, _O, _C = 10000, 128, 128, 2, 10
_RB = 400
_NBA = _N // _RB
_W = 2048
_NW = 5
_G = 5
_GR = _G * _RB
_NG = _NBA // _G
_ZC = 5120


def _body_a(adj_ref, x_ref, W1_ref, b1_ref, W2_ref, b2_ref,
            p_ref, zp_ref, u_ref, pacc_ref, zacc_ref):
    i = pl.program_id(0)

    @pl.when(i == 0)
    def _():
        u_ref[...] = jnp.dot(x_ref[...], W1_ref[...],
                             preferred_element_type=jnp.float32)
        pacc_ref[...] = jnp.zeros_like(pacc_ref)

    hk = jnp.maximum(
        jnp.dot(adj_ref[...], u_ref[...], preferred_element_type=jnp.float32)
        + b1_ref[...], 0.0)
    pk = jnp.dot(hk, W2_ref[...], preferred_element_type=jnp.float32)
    pacc_ref[pl.ds(i * _RB, _RB), :] = pk
    p_ref[...] = pk

    zacc_ref[...] = jnp.broadcast_to(b2_ref[...], (_RB, _O))
    for lo in range(0, _N, _ZC):
        hi = min(_N, lo + _ZC)

        @pl.when(lo < (i + 1) * _RB)
        def _(lo=lo, hi=hi):
            zacc_ref[...] += jnp.dot(adj_ref[:, lo:hi], pacc_ref[lo:hi, :],
                                     preferred_element_type=jnp.float32)
    zp_ref[...] = zacc_ref[...]


def _body_b(gs_ref, cs_ref, adj_ref, p_ref, zp_ref, muT_ref, z_ref, q_ref):
    t = pl.program_id(0)
    g = gs_ref[t]
    c = cs_ref[t]
    first = jnp.logical_or(t == 0, gs_ref[jnp.maximum(t - 1, 0)] != g)
    last = c == _NW - 1

    valid = _N - c * _W
    cols = jax.lax.broadcasted_iota(jnp.int32, (1, _W), 1)
    cols_ok = cols < valid
    rows = jax.lax.broadcasted_iota(jnp.int32, (_W, 1), 0)
    rows_ok = rows < valid
    p_blk = p_ref[...]

    @pl.when(first)
    def _():
        z_ref[...] = zp_ref[...]

    for s in range(_G):
        k = g * _G + s
        thresh = (k + 1) * _RB - c * _W
        pm = jnp.where((rows >= thresh) & rows_ok, p_blk, 0.0)
        am_s = jnp.where(cols_ok, adj_ref[s * _RB:(s + 1) * _RB, :], 0.0)
        z_ref[s * _RB:(s + 1) * _RB, :] += jnp.dot(
            am_s, pm, preferred_element_type=jnp.float32)

    @pl.when(last)
    def _():
        z = z_ref[...]
        d2 = ((z[:, 0:1] - muT_ref[0:1, :]) ** 2
              + (z[:, 1:2] - muT_ref[1:2, :]) ** 2)
        qr = 1.0 / (1.0 + d2 * 2.0 + 1e-6)
        qr = qr * jnp.sqrt(qr)
        q_ref[...] = qr / jnp.sum(qr, axis=1, keepdims=True)


def kernel(x, adj, W1, b1, W2, b2, mu):
    b1r = b1.reshape(1, _H)
    b2r = b2.reshape(1, _O)
    muT = mu.T

    p, zp = pl.pallas_call(
        _body_a,
        grid=(_NBA,),
        in_specs=[
            pl.BlockSpec((_RB, _N), lambda i: (i, 0)),
            pl.BlockSpec((_N, _D), lambda i: (0, 0)),
            pl.BlockSpec((_D, _H), lambda i: (0, 0)),
            pl.BlockSpec((1, _H), lambda i: (0, 0)),
            pl.BlockSpec((_H, _O), lambda i: (0, 0)),
            pl.BlockSpec((1, _O), lambda i: (0, 0)),
        ],
        out_specs=[
            pl.BlockSpec((_RB, _O), lambda i: (i, 0)),
            pl.BlockSpec((_RB, _O), lambda i: (i, 0)),
        ],
        out_shape=[
            jax.ShapeDtypeStruct((_N, _O), jnp.float32),
            jax.ShapeDtypeStruct((_N, _O), jnp.float32),
        ],
        scratch_shapes=[
            pltpu.VMEM((_N, _D), jnp.float32),
            pltpu.VMEM((_N, _O), jnp.float32),
            pltpu.VMEM((_RB, _O), jnp.float32),
        ],
    )(adj, x, W1, b1r, W2, b2r)

    gs_l, cs_l = [], []
    for g in range(_NG):
        for c in range(min(_NW - 1, (g * _G + 1) * _RB // _W), _NW):
            gs_l.append(g)
            cs_l.append(c)
    nt = len(gs_l)
    gs = jnp.array(gs_l, dtype=jnp.int32)
    cs = jnp.array(cs_l, dtype=jnp.int32)

    grid_spec = pltpu.PrefetchScalarGridSpec(
        num_scalar_prefetch=2,
        grid=(nt,),
        in_specs=[
            pl.BlockSpec((_GR, _W), lambda t, gs, cs: (gs[t], cs[t])),
            pl.BlockSpec((_W, _O), lambda t, gs, cs: (cs[t], 0)),
            pl.BlockSpec((_GR, _O), lambda t, gs, cs: (gs[t], 0)),
            pl.BlockSpec((_O, _C), lambda t, gs, cs: (0, 0)),
        ],
        out_specs=[
            pl.BlockSpec((_GR, _O), lambda t, gs, cs: (gs[t], 0)),
            pl.BlockSpec((_GR, _C), lambda t, gs, cs: (gs[t], 0)),
        ],
    )
    z, q = pl.pallas_call(
        _body_b,
        grid_spec=grid_spec,
        out_shape=[
            jax.ShapeDtypeStruct((_N, _O), jnp.float32),
            jax.ShapeDtypeStruct((_N, _C), jnp.float32),
        ],
    )(gs, cs, adj, p, zp, muT)
    return (z, q)


# speedup vs baseline: 1.0296x; 1.0296x over previous
import jax
import jax.numpy as jnp
from jax.experimental import pallas as pl
from jax.experimental.pallas import tpu as pltpu

_N, _D, _H, _O, _C = 10000, 128, 128, 2, 10
_RB = 400
_NBA = _N // _RB
_W = 2048
_NW = 5
_G = 5
_GR = _G * _RB
_NG = _NBA // _G
_ZC = 5120


def _body_a(adj_ref, x_ref, W1_ref, b1_ref, W2_ref, b2_ref,
            p_ref, zp_ref, ua_ref):
    i = pl.program_id(0)

    @pl.when(i == 0)
    def _():
        ua_ref[:, :_D] = jnp.dot(x_ref[...], W1_ref[...],
                                 preferred_element_type=jnp.float32)
        ua_ref[:, _D:] = jnp.zeros((_N, _O), jnp.float32)

    s_aug = jnp.dot(adj_ref[...], ua_ref[...],
                    preferred_element_type=jnp.float32)
    hk = jnp.maximum(s_aug[:, :_D] + b1_ref[...], 0.0)
    pk = jnp.dot(hk, W2_ref[...], preferred_element_type=jnp.float32)
    ua_ref[pl.ds(i * _RB, _RB), _D:] = pk
    p_ref[...] = pk

    zp_ref[...] = s_aug[:, _D:] + b2_ref[...]
    for c in range(_NW):
        lo = c * _W
        hi = min(_N, lo + _W)

        @pl.when((lo < (i + 1) * _RB) & (hi > i * _RB))
        def _(lo=lo, hi=hi):
            rows = lo + jax.lax.broadcasted_iota(jnp.int32, (hi - lo, 1), 0)
            pm = jnp.where((rows >= i * _RB) & (rows < (i + 1) * _RB),
                           ua_ref[lo:hi, _D:], 0.0)
            zp_ref[...] += jnp.dot(adj_ref[:, lo:hi], pm,
                                   preferred_element_type=jnp.float32)


def _body_b(gs_ref, cs_ref, adj_ref, p_ref, zp_ref, muT_ref, z_ref, q_ref):
    t = pl.program_id(0)
    g = gs_ref[t]
    c = cs_ref[t]
    first = jnp.logical_or(t == 0, gs_ref[jnp.maximum(t - 1, 0)] != g)
    last = c == _NW - 1

    valid = _N - c * _W
    cols = jax.lax.broadcasted_iota(jnp.int32, (1, _W), 1)
    cols_ok = cols < valid
    rows = jax.lax.broadcasted_iota(jnp.int32, (_W, 1), 0)
    rows_ok = rows < valid
    p_blk = p_ref[...]

    @pl.when(first)
    def _():
        z_ref[...] = zp_ref[...]

    for s in range(_G):
        k = g * _G + s
        thresh = (k + 1) * _RB - c * _W
        pm = jnp.where((rows >= thresh) & rows_ok, p_blk, 0.0)
        am_s = jnp.where(cols_ok, adj_ref[s * _RB:(s + 1) * _RB, :], 0.0)
        z_ref[s * _RB:(s + 1) * _RB, :] += jnp.dot(
            am_s, pm, preferred_element_type=jnp.float32)

    @pl.when(last)
    def _():
        z = z_ref[...]
        d2 = ((z[:, 0:1] - muT_ref[0:1, :]) ** 2
              + (z[:, 1:2] - muT_ref[1:2, :]) ** 2)
        qr = 1.0 / (1.0 + d2 * 2.0 + 1e-6)
        qr = qr * jnp.sqrt(qr)
        q_ref[...] = qr / jnp.sum(qr, axis=1, keepdims=True)


def kernel(x, adj, W1, b1, W2, b2, mu):
    b1r = b1.reshape(1, _H)
    b2r = b2.reshape(1, _O)
    muT = mu.T

    p, zp = pl.pallas_call(
        _body_a,
        grid=(_NBA,),
        in_specs=[
            pl.BlockSpec((_RB, _N), lambda i: (i, 0)),
            pl.BlockSpec((_N, _D), lambda i: (0, 0)),
            pl.BlockSpec((_D, _H), lambda i: (0, 0)),
            pl.BlockSpec((1, _H), lambda i: (0, 0)),
            pl.BlockSpec((_H, _O), lambda i: (0, 0)),
            pl.BlockSpec((1, _O), lambda i: (0, 0)),
        ],
        out_specs=[
            pl.BlockSpec((_RB, _O), lambda i: (i, 0)),
            pl.BlockSpec((_RB, _O), lambda i: (i, 0)),
        ],
        out_shape=[
            jax.ShapeDtypeStruct((_N, _O), jnp.float32),
            jax.ShapeDtypeStruct((_N, _O), jnp.float32),
        ],
        scratch_shapes=[
            pltpu.VMEM((_N, _D + _O), jnp.float32),
        ],
    )(adj, x, W1, b1r, W2, b2r)

    gs_l, cs_l = [], []
    for g in range(_NG):
        for c in range(min(_NW - 1, (g * _G + 1) * _RB // _W), _NW):
            gs_l.append(g)
            cs_l.append(c)
    nt = len(gs_l)
    gs = jnp.array(gs_l, dtype=jnp.int32)
    cs = jnp.array(cs_l, dtype=jnp.int32)

    grid_spec = pltpu.PrefetchScalarGridSpec(
        num_scalar_prefetch=2,
        grid=(nt,),
        in_specs=[
            pl.BlockSpec((_GR, _W), lambda t, gs, cs: (gs[t], cs[t])),
            pl.BlockSpec((_W, _O), lambda t, gs, cs: (cs[t], 0)),
            pl.BlockSpec((_GR, _O), lambda t, gs, cs: (gs[t], 0)),
            pl.BlockSpec((_O, _C), lambda t, gs, cs: (0, 0)),
        ],
        out_specs=[
            pl.BlockSpec((_GR, _O), lambda t, gs, cs: (gs[t], 0)),
            pl.BlockSpec((_GR, _C), lambda t, gs, cs: (gs[t], 0)),
        ],
    )
    z, q = pl.pallas_call(
        _body_b,
        grid_spec=grid_spec,
        out_shape=[
            jax.ShapeDtypeStruct((_N, _O), jnp.float32),
            jax.ShapeDtypeStruct((_N, _C), jnp.float32),
        ],
    )(gs, cs, adj, p, zp, muT)
    return (z, q)


# speedup vs baseline: 1.0352x; 1.0055x over previous
import jax
import jax.numpy as jnp
from jax.experimental import pallas as pl
from jax.experimental.pallas import tpu as pltpu

_N, _D, _H, _O, _C = 10000, 128, 128, 2, 10
_RB = 400
_NBA = _N // _RB
_W = 2048
_NW = 5
_G = 5
_GR = _G * _RB
_NG = _NBA // _G
_ZC = 5120


def _body_a(adj_ref, x_ref, W1_ref, b1_ref, W2_ref, b2_ref,
            pz_ref, ua_ref):
    i = pl.program_id(0)

    @pl.when(i == 0)
    def _():
        ua_ref[:, :_D] = jnp.dot(x_ref[...], W1_ref[...],
                                 preferred_element_type=jnp.float32)
        ua_ref[:, _D:] = jnp.zeros((_N, _O), jnp.float32)

    s_aug = jnp.dot(adj_ref[...], ua_ref[...],
                    preferred_element_type=jnp.float32)
    hk = jnp.maximum(s_aug[:, :_D] + b1_ref[...], 0.0)
    pk = jnp.dot(hk, W2_ref[...], preferred_element_type=jnp.float32)
    ua_ref[pl.ds(i * _RB, _RB), _D:] = pk
    pz_ref[:, :_O] = pk

    pz_ref[:, _O:] = s_aug[:, _D:] + b2_ref[...]
    for c in range(_NW):
        lo = c * _W
        hi = min(_N, lo + _W)

        @pl.when((lo < (i + 1) * _RB) & (hi > i * _RB))
        def _(lo=lo, hi=hi):
            rows = lo + jax.lax.broadcasted_iota(jnp.int32, (hi - lo, 1), 0)
            pm = jnp.where((rows >= i * _RB) & (rows < (i + 1) * _RB),
                           ua_ref[lo:hi, _D:], 0.0)
            pz_ref[:, _O:] += jnp.dot(adj_ref[:, lo:hi], pm,
                                      preferred_element_type=jnp.float32)


def _body_b(gs_ref, cs_ref, adj_ref, p_ref, zp_ref, muT_ref, z_ref, q_ref):
    t = pl.program_id(0)
    g = gs_ref[t]
    c = cs_ref[t]
    first = jnp.logical_or(t == 0, gs_ref[jnp.maximum(t - 1, 0)] != g)
    last = c == _NW - 1

    valid = _N - c * _W
    cols = jax.lax.broadcasted_iota(jnp.int32, (1, _W), 1)
    cols_ok = cols < valid
    rows = jax.lax.broadcasted_iota(jnp.int32, (_W, 1), 0)
    rows_ok = rows < valid
    p_blk = p_ref[:, :_O]

    @pl.when(first)
    def _():
        z_ref[...] = zp_ref[:, _O:]

    for s in range(_G):
        k = g * _G + s
        thresh = (k + 1) * _RB - c * _W
        pm = jnp.where((rows >= thresh) & rows_ok, p_blk, 0.0)
        am_s = jnp.where(cols_ok, adj_ref[s * _RB:(s + 1) * _RB, :], 0.0)
        z_ref[s * _RB:(s + 1) * _RB, :] += jnp.dot(
            am_s, pm, preferred_element_type=jnp.float32)

    @pl.when(last)
    def _():
        z = z_ref[...]
        d2 = ((z[:, 0:1] - muT_ref[0:1, :]) ** 2
              + (z[:, 1:2] - muT_ref[1:2, :]) ** 2)
        qr = 1.0 / (1.0 + d2 * 2.0 + 1e-6)
        qr = qr * jnp.sqrt(qr)
        q_ref[...] = qr / jnp.sum(qr, axis=1, keepdims=True)


def kernel(x, adj, W1, b1, W2, b2, mu):
    b1r = b1.reshape(1, _H)
    b2r = b2.reshape(1, _O)
    muT = mu.T

    pz = pl.pallas_call(
        _body_a,
        grid=(_NBA,),
        in_specs=[
            pl.BlockSpec((_RB, _N), lambda i: (i, 0)),
            pl.BlockSpec((_N, _D), lambda i: (0, 0)),
            pl.BlockSpec((_D, _H), lambda i: (0, 0)),
            pl.BlockSpec((1, _H), lambda i: (0, 0)),
            pl.BlockSpec((_H, _O), lambda i: (0, 0)),
            pl.BlockSpec((1, _O), lambda i: (0, 0)),
        ],
        out_specs=pl.BlockSpec((_RB, 2 * _O), lambda i: (i, 0)),
        out_shape=jax.ShapeDtypeStruct((_N, 2 * _O), jnp.float32),
        scratch_shapes=[
            pltpu.VMEM((_N, _D + _O), jnp.float32),
        ],
    )(adj, x, W1, b1r, W2, b2r)

    gs_l, cs_l = [], []
    for g in range(_NG):
        for c in range(min(_NW - 1, (g * _G + 1) * _RB // _W), _NW):
            gs_l.append(g)
            cs_l.append(c)
    nt = len(gs_l)
    gs = jnp.array(gs_l, dtype=jnp.int32)
    cs = jnp.array(cs_l, dtype=jnp.int32)

    grid_spec = pltpu.PrefetchScalarGridSpec(
        num_scalar_prefetch=2,
        grid=(nt,),
        in_specs=[
            pl.BlockSpec((_GR, _W), lambda t, gs, cs: (gs[t], cs[t])),
            pl.BlockSpec((_W, 2 * _O), lambda t, gs, cs: (cs[t], 0)),
            pl.BlockSpec((_GR, 2 * _O), lambda t, gs, cs: (gs[t], 0)),
            pl.BlockSpec((_O, _C), lambda t, gs, cs: (0, 0)),
        ],
        out_specs=[
            pl.BlockSpec((_GR, _O), lambda t, gs, cs: (gs[t], 0)),
            pl.BlockSpec((_GR, _C), lambda t, gs, cs: (gs[t], 0)),
        ],
    )
    z, q = pl.pallas_call(
        _body_b,
        grid_spec=grid_spec,
        out_shape=[
            jax.ShapeDtypeStruct((_N, _O), jnp.float32),
            jax.ShapeDtypeStruct((_N, _C), jnp.float32),
        ],
    )(gs, cs, adj, pz, pz, muT)
    return (z, q)
